# Initial kernel scaffold; baseline (speedup 1.0000x reference)
import jax
import jax.numpy as jnp
from jax.experimental import pallas as pl


def kernel(x):
    raise NotImplementedError("write your pallas kernel here")


# speedup vs baseline: 9.8319x; 9.8319x over previous
import jax
import jax.numpy as jnp
from jax.experimental import pallas as pl

_TEMP = 2.0 / 3.0
_LIMIT_A = -0.1
_LIMIT_B = 1.1
_K = 256


def _topk_mask_kernel(x_ref, mask_ref, s_ref):
    xb = x_ref[...]
    rows, cols = xb.shape

    u = (xb + 1.0) * (1.0 / _TEMP)
    z = jax.nn.sigmoid(u)
    r = z * (_LIMIT_B - _LIMIT_A) + _LIMIT_A
    s_ref[...] = jnp.clip(r, 0.0, 1.0)

    key = jax.lax.bitcast_convert_type(z, jnp.int32)

    lo = jnp.zeros((rows, 1), dtype=jnp.int32)
    hi = jnp.full((rows, 1), 0x3F800000, dtype=jnp.int32)

    def vbody(_, carry):
        lo, hi = carry
        mid = (lo + hi) >> 1
        cnt = jnp.sum((key > mid).astype(jnp.int32), axis=1, keepdims=True)
        pred = cnt < _K
        hi = jnp.where(pred, mid, hi)
        lo = jnp.where(pred, lo, mid + 1)
        return lo, hi

    lo, hi = jax.lax.fori_loop(0, 30, vbody, (lo, hi))
    t = lo

    cnt_gt = jnp.sum((key > t).astype(jnp.int32), axis=1, keepdims=True)
    m = _K - cnt_gt
    tie = key == t
    col = jax.lax.broadcasted_iota(jnp.int32, (rows, cols), 1)

    ilo = jnp.zeros((rows, 1), dtype=jnp.int32)
    ihi = jnp.full((rows, 1), cols - 1, dtype=jnp.int32)

    def ibody(_, carry):
        ilo, ihi = carry
        mid = (ilo + ihi) >> 1
        cnt = jnp.sum((tie & (col <= mid)).astype(jnp.int32), axis=1,
                      keepdims=True)
        pred = cnt >= m
        ihi = jnp.where(pred, mid, ihi)
        ilo = jnp.where(pred, ilo, mid + 1)
        return ilo, ihi

    ilo, ihi = jax.lax.fori_loop(0, 15, ibody, (ilo, ihi))

    sel = (key > t) | (tie & (col <= ilo))
    mask_ref[...] = sel.astype(jnp.float32)


def kernel(x):
    n_rows, n_cols = x.shape
    block_rows = 8
    grid = (n_rows // block_rows,)
    out_shape = (
        jax.ShapeDtypeStruct((n_rows, n_cols), jnp.float32),
        jax.ShapeDtypeStruct((n_rows, n_cols), jnp.float32),
    )
    blk = pl.BlockSpec((block_rows, n_cols), lambda i: (i, 0))
    mask, s = pl.pallas_call(
        _topk_mask_kernel,
        grid=grid,
        in_specs=[blk],
        out_specs=(blk, blk),
        out_shape=out_shape,
    )(x)
    return (mask, s)


# speedup vs baseline: 21.6030x; 2.1972x over previous
import jax
import jax.numpy as jnp
from jax.experimental import pallas as pl

_TEMP = 2.0 / 3.0
_LIMIT_A = -0.1
_LIMIT_B = 1.1
_K = 256


def _rowsum(v):
    n = v.shape[1]
    c = n // 4
    p = [
        jnp.sum(v[:, i * c:(i + 1) * c].astype(jnp.int32), axis=1,
                keepdims=True)
        for i in range(4)
    ]
    return (p[0] + p[1]) + (p[2] + p[3])


def _topk_mask_kernel(x_ref, mask_ref, s_ref):
    xb = x_ref[...]
    rows, cols = xb.shape

    u = (xb + 1.0) * (1.0 / _TEMP)
    z = jax.nn.sigmoid(u)
    r = z * (_LIMIT_B - _LIMIT_A) + _LIMIT_A
    s_ref[...] = jnp.clip(r, 0.0, 1.0)

    key = jax.lax.bitcast_convert_type(z, jnp.int32)

    lo = jnp.zeros((rows, 1), dtype=jnp.int32)
    hi = jnp.full((rows, 1), 0x3F800000, dtype=jnp.int32)

    def vbody(_, carry):
        lo, hi = carry
        mid = (lo + hi) >> 1
        cnt = _rowsum(key > mid)
        pred = cnt < _K
        hi = jnp.where(pred, mid, hi)
        lo = jnp.where(pred, lo, mid + 1)
        return lo, hi

    lo, hi = jax.lax.fori_loop(0, 30, vbody, (lo, hi))
    t = lo

    cnt_gt = _rowsum(key > t)
    m = _K - cnt_gt
    tie = key == t
    col = jax.lax.broadcasted_iota(jnp.int32, (rows, cols), 1)

    ilo = jnp.zeros((rows, 1), dtype=jnp.int32)
    ihi = jnp.full((rows, 1), cols - 1, dtype=jnp.int32)

    def ibody(_, carry):
        ilo, ihi = carry
        mid = (ilo + ihi) >> 1
        cnt = _rowsum(tie & (col <= mid))
        pred = cnt >= m
        ihi = jnp.where(pred, mid, ihi)
        ilo = jnp.where(pred, ilo, mid + 1)
        return ilo, ihi

    ilo, ihi = jax.lax.fori_loop(0, 15, ibody, (ilo, ihi))

    sel = (key > t) | (tie & (col <= ilo))
    mask_ref[...] = sel.astype(jnp.float32)


def kernel(x):
    n_rows, n_cols = x.shape
    block_rows = 32
    grid = (n_rows // block_rows,)
    out_shape = (
        jax.ShapeDtypeStruct((n_rows, n_cols), jnp.float32),
        jax.ShapeDtypeStruct((n_rows, n_cols), jnp.float32),
    )
    blk = pl.BlockSpec((block_rows, n_cols), lambda i: (i, 0))
    mask, s = pl.pallas_call(
        _topk_mask_kernel,
        grid=grid,
        in_specs=[blk],
        out_specs=(blk, blk),
        out_shape=out_shape,
    )(x)
    return (mask, s)
